# Initial kernel scaffold; baseline (speedup 1.0000x reference)
import jax
import jax.numpy as jnp
from jax.experimental import pallas as pl


def kernel(x, edge_index, node_class, batch, node_emb, w_in1, b_in1, w_in2, b_in2, gw1, gb1, gw2, gb2, gw3, gb3, wp1, bp1, wp2, bp2):
    raise NotImplementedError("write your pallas kernel here")


# speedup vs baseline: 9.2797x; 9.2797x over previous
import functools

import jax
import jax.numpy as jnp
from jax import lax
from jax.experimental import pallas as pl
from jax.experimental.pallas import tpu as pltpu
from jax.experimental.pallas import tpu_sc as plsc

N = 10000
E = 320000
D_IN = 128
HID = 256
HALF = 128
QTR = 64
PROJ = 128
NCLS = 1000
EMB = 64
G = 64

NC = 2
NS = 16
NPAD = 10240
EPAD = 323584
CH = 128
TPE = EPAD // NS
NCHUNK = TPE // CH
DPE = EPAD // (NC * NS)
DCHUNK = DPE // CH
RPT = NPAD // NS
ACC_R = 10112
ABLK = ACC_R // CH
GP = G + 1

_mesh = plsc.VectorSubcoreMesh(core_axis_name="c", subcore_axis_name="s")


def _zero_block(zb, rows, width):
    zv = jnp.zeros((16,), jnp.float32)

    def body(i, _):
        for j in range(width // 16):
            zb[i, pl.ds(j * 16, 16)] = zv
        return 0

    lax.fori_loop(0, rows, body, 0)


def _sc_gather_deg(t_hbm, ncls_hbm, dst_hbm, erows_hbm, degp_hbm,
                   idx0, idx1, rows0, rows1, ones, zb, dega, sem0, sem1):
    c = lax.axis_index("c")
    s = lax.axis_index("s")

    one16 = jnp.ones((16,), jnp.float32)
    for j in range(CH // 16):
        ones[pl.ds(j * 16, 16)] = one16
    zv = jnp.zeros((16,), jnp.float32)

    def zbody(i, _):
        zb[pl.ds(i * 16, 16)] = zv
        return 0

    lax.fori_loop(0, RPT // 16, zbody, 0)

    nbase = s * RPT
    bufs = ((idx0, rows0, sem0), (idx1, rows1, sem1))

    def start(k, slot):
        i, r, sm = bufs[slot]
        pltpu.sync_copy(ncls_hbm.at[pl.ds(nbase + k * CH, CH)], i)
        return pltpu.async_copy(t_hbm.at[c].at[i], r, sm)

    d = start(0, 0)
    for k in range(5):
        nxt = None
        if k + 1 < 5:
            nxt = start(k + 1, (k + 1) % 2)
        d.wait()
        _, r, _ = bufs[k % 2]
        pltpu.sync_copy(r, erows_hbm.at[c, pl.ds(nbase + k * CH, CH)])
        d = nxt

    pltpu.sync_copy(zb, dega.at[pl.ds(s * RPT, RPT)])
    plsc.subcore_barrier()

    ebase = (c * NS + s) * DPE

    def dbody(k, _):
        pltpu.sync_copy(dst_hbm.at[pl.ds(ebase + k * CH, CH)], idx0)
        pltpu.sync_copy(ones, dega.at[idx0], add=True)
        return 0

    lax.fori_loop(0, DCHUNK, dbody, 0)
    plsc.subcore_barrier()
    pltpu.sync_copy(dega.at[pl.ds(s * RPT, RPT)], degp_hbm.at[c, pl.ds(s * RPT, RPT)])


def _sc_scatter(g_hbm, src_hbm, dstc_hbm, s_hbm,
                idx0, idx1, idxd, rows0, rows1, zb, acc, sem0, sem1):

    c = lax.axis_index("c")
    s = lax.axis_index("s")

    _zero_block(zb, CH, HALF)
    ebase = s * TPE

    for k in range(5):
        blk = s + NS * k

        @pl.when(blk < ABLK)
        def _():
            pltpu.sync_copy(zb, acc.at[pl.ds(blk * CH, CH)])

    plsc.subcore_barrier()

    gtab = g_hbm.at[c]

    def start(chunk, ibuf, rbuf, sm):
        pltpu.sync_copy(src_hbm.at[pl.ds(ebase + chunk * CH, CH)], ibuf)
        return pltpu.async_copy(gtab.at[ibuf], rbuf, sm)

    def scat(chunk, rbuf):
        pltpu.sync_copy(dstc_hbm.at[pl.ds(ebase + chunk * CH, CH)], idxd)
        pltpu.sync_copy(rbuf, acc.at[idxd], add=True)

    start(0, idx0, rows0, sem0)

    def ebody(gidx, _):
        a = 2 * gidx
        start(a + 1, idx1, rows1, sem1)
        pltpu.make_async_copy(gtab.at[idx0], rows0, sem0).wait()
        scat(a, rows0)
        nxt = jnp.minimum(a + 2, NCHUNK - 1)
        start(nxt, idx0, rows0, sem0)
        pltpu.make_async_copy(gtab.at[idx1], rows1, sem1).wait()
        scat(a + 1, rows1)
        return 0

    lax.fori_loop(0, NCHUNK // 2, ebody, 0)
    pltpu.make_async_copy(gtab.at[idx0], rows0, sem0).wait()
    plsc.subcore_barrier()

    for k in range(5):
        blk = s + NS * k

        @pl.when(blk < ABLK)
        def _():
            pltpu.sync_copy(acc.at[pl.ds(blk * CH, CH)],
                            s_hbm.at[c, pl.ds(blk * CH, CH)])


def _sc_pool(s_hbm, g_hbm, dinv_hbm, batch_hbm, bias_hbm,
             sump_hbm, maxp_hbm, cntp_hbm,
             rows0, gst, dinvb, batchb, biasb, sacc, macc, cacc):
    c = lax.axis_index("c")
    s = lax.axis_index("s")

    pltpu.sync_copy(bias_hbm.at[c], biasb)
    _zero_block(sacc, GP, HALF)
    ninf = jnp.full((16,), -jnp.inf, jnp.float32)
    zv16 = jnp.zeros((16,), jnp.float32)

    def ibody(i, _):
        for j in range(HALF // 16):
            macc[i, pl.ds(j * 16, 16)] = ninf
        cacc[i, pl.ds(0, 16)] = zv16
        return 0

    lax.fori_loop(0, GP, ibody, 0)
    e0 = jnp.where(lax.iota(jnp.int32, 16) == 0, 1.0, 0.0)

    for k in range(RPT // CH):
        off = s * RPT + k * CH
        pltpu.sync_copy(dinv_hbm.at[0, pl.ds(off, CH)], dinvb)
        pltpu.sync_copy(batch_hbm.at[pl.ds(off, CH)], batchb)
        pltpu.sync_copy(s_hbm.at[c, pl.ds(off, CH)], rows0)
        pltpu.sync_copy(g_hbm.at[c, pl.ds(off, CH)], gst)

        def qbody(q, _):
            dv = dinvb[pl.ds(q * 16, 16)]
            bv = batchb[pl.ds(q * 16, 16)]
            for l in range(16):
                n = q * 16 + l
                dn = dv[l]
                b = bv[l]
                cacc[b, pl.ds(0, 16)] += e0
                for j in range(HALF // 16):
                    srow = rows0[n, pl.ds(j * 16, 16)]
                    grow = gst[n, pl.ds(j * 16, 16)]
                    h = jnp.maximum(dn * (srow + grow) + biasb[pl.ds(j * 16, 16)], 0.0)
                    sacc[b, pl.ds(j * 16, 16)] += h
                    macc[b, pl.ds(j * 16, 16)] = jnp.maximum(macc[b, pl.ds(j * 16, 16)], h)
            return 0

        lax.fori_loop(0, CH // 16, qbody, 0)

    pltpu.sync_copy(sacc, sump_hbm.at[c, s])
    pltpu.sync_copy(macc, maxp_hbm.at[c, s])
    pltpu.sync_copy(cacc, cntp_hbm.at[c, s])


_sc_scatter_only = pl.kernel(
    _sc_scatter,
    out_type=jax.ShapeDtypeStruct((NC, NPAD, HALF), jnp.float32),
    mesh=_mesh,
    scratch_types=[
        pltpu.VMEM((CH,), jnp.int32),
        pltpu.VMEM((CH,), jnp.int32),
        pltpu.VMEM((CH,), jnp.int32),
        pltpu.VMEM((CH, HALF), jnp.float32),
        pltpu.VMEM((CH, HALF), jnp.float32),
        pltpu.VMEM((CH, HALF), jnp.float32),
        pltpu.VMEM_SHARED((ACC_R, HALF), jnp.float32),
        pltpu.SemaphoreType.DMA, pltpu.SemaphoreType.DMA,
    ],
)

_sc_pool_call = pl.kernel(
    _sc_pool,
    out_type=(jax.ShapeDtypeStruct((NC, NS, GP, HALF), jnp.float32),
              jax.ShapeDtypeStruct((NC, NS, GP, HALF), jnp.float32),
              jax.ShapeDtypeStruct((NC, NS, GP, 16), jnp.float32)),
    mesh=_mesh,
    scratch_types=[
        pltpu.VMEM((CH, HALF), jnp.float32),
        pltpu.VMEM((CH, HALF), jnp.float32),
        pltpu.VMEM((CH,), jnp.float32),
        pltpu.VMEM((CH,), jnp.int32),
        pltpu.VMEM((HALF,), jnp.float32),
        pltpu.VMEM((GP, HALF), jnp.float32),
        pltpu.VMEM((GP, HALF), jnp.float32),
        pltpu.VMEM((GP, 16), jnp.float32),
    ],
)

_sc_gather_deg_call = pl.kernel(
    _sc_gather_deg,
    out_type=(jax.ShapeDtypeStruct((NC, NPAD, HALF), jnp.float32),
              jax.ShapeDtypeStruct((NC, NPAD), jnp.float32)),
    mesh=_mesh,
    scratch_types=[
        pltpu.VMEM((CH,), jnp.int32),
        pltpu.VMEM((CH,), jnp.int32),
        pltpu.VMEM((CH, HALF), jnp.float32),
        pltpu.VMEM((CH, HALF), jnp.float32),
        pltpu.VMEM((CH,), jnp.float32),
        pltpu.VMEM((RPT,), jnp.float32),
        pltpu.VMEM_SHARED((NPAD,), jnp.float32),
        pltpu.SemaphoreType.DMA, pltpu.SemaphoreType.DMA,
    ],
)


def _split2(g_ref, gg):
    g_ref[0] = gg[:, :HALF]
    g_ref[1] = gg[:, HALF:]


def _cat2(ref):
    return jnp.concatenate([ref[0], ref[1]], axis=1)


def _tc_emb_body(ne_ref, w_ref, out_ref):
    t = jnp.dot(ne_ref[...], w_ref[...], preferred_element_type=jnp.float32)
    out_ref[0] = t[:, :HALF]
    out_ref[1] = t[:, HALF:]


def _tc_in_body(x_ref, er_ref, degp_ref, w1_ref, b1_ref, w2_ref, b2_ref,
                gw_ref, g_ref, dinv_ref):
    deg = degp_ref[0] + degp_ref[1] + 1.0
    dinv = lax.rsqrt(deg)
    h = jnp.concatenate([er_ref[0], er_ref[1]], axis=1)
    h = h + jnp.dot(x_ref[...], w1_ref[...], preferred_element_type=jnp.float32)
    h = jnp.maximum(h + b1_ref[...], 0.0)
    h = jnp.dot(h, w2_ref[...], preferred_element_type=jnp.float32) + b2_ref[...]
    h = jnp.maximum(h, 0.0)
    g = dinv[:, None] * jnp.dot(h, gw_ref[...], preferred_element_type=jnp.float32)
    _split2(g_ref, g)
    dinv_ref[0, :] = dinv


def _tc_layer_body(s_ref, g_ref, dinv_ref, b_ref, w_ref, out_ref):
    dinv = dinv_ref[0, :][:, None]
    h = _cat2(s_ref) + _cat2(g_ref)
    h = jnp.maximum(dinv * h + b_ref[...], 0.0)
    g = dinv * jnp.dot(h, w_ref[...], preferred_element_type=jnp.float32)
    _split2(out_ref, g)


def _tc_head_body(sump_ref, maxp_ref, cntp_ref, wp1_ref, bp1_ref,
                  wp2_ref, bp2_ref, z_ref):
    cnt = jnp.sum(cntp_ref[0], axis=(0, 2))[:G]
    ssum = jnp.sum(sump_ref[...], axis=1)
    mean = jnp.concatenate([ssum[0], ssum[1]], axis=1)[:G] / jnp.clip(cnt, 1.0)[:, None]
    mxx = jnp.max(maxp_ref[...], axis=1)
    mx = jnp.concatenate([mxx[0], mxx[1]], axis=1)[:G]
    hg = jnp.concatenate([mean, mx], axis=1)
    z = jnp.maximum(jnp.dot(hg, wp1_ref[...], preferred_element_type=jnp.float32)
                    + bp1_ref[...], 0.0)
    z = jnp.dot(z, wp2_ref[...], preferred_element_type=jnp.float32) + bp2_ref[...]
    nrm = jnp.sqrt(jnp.sum(z * z, axis=1, keepdims=True))
    z_ref[...] = z / jnp.clip(nrm, 1e-12)


_BM = 1024
_GRID = NPAD // _BM


def _rows_spec():
    return pl.BlockSpec((NC, _BM, HALF), lambda i: (0, i, 0))


_tc_emb = pl.pallas_call(
    _tc_emb_body,
    out_shape=jax.ShapeDtypeStruct((NC, NCLS, HALF), jnp.float32),
    in_specs=[pl.BlockSpec((NCLS, EMB), lambda: (0, 0)),
              pl.BlockSpec((EMB, HID), lambda: (0, 0))],
    out_specs=pl.BlockSpec((NC, NCLS, HALF), lambda: (0, 0, 0)),
)

_tc_in = pl.pallas_call(
    _tc_in_body,
    grid=(_GRID,),
    out_shape=(jax.ShapeDtypeStruct((NC, NPAD, HALF), jnp.float32),
               jax.ShapeDtypeStruct((1, NPAD), jnp.float32)),
    in_specs=[pl.BlockSpec((_BM, D_IN), lambda i: (i, 0)),
              pl.BlockSpec((NC, _BM, HALF), lambda i: (0, i, 0)),
              pl.BlockSpec((NC, _BM), lambda i: (0, i)),
              pl.BlockSpec((D_IN, HID), lambda i: (0, 0)),
              pl.BlockSpec((1, HID), lambda i: (0, 0)),
              pl.BlockSpec((HID, HID), lambda i: (0, 0)),
              pl.BlockSpec((1, HID), lambda i: (0, 0)),
              pl.BlockSpec((HID, HID), lambda i: (0, 0))],
    out_specs=(_rows_spec(),
               pl.BlockSpec((1, _BM), lambda i: (0, i))),
)

_tc_layer = pl.pallas_call(
    _tc_layer_body,
    grid=(_GRID,),
    out_shape=jax.ShapeDtypeStruct((NC, NPAD, HALF), jnp.float32),
    in_specs=[_rows_spec(),
              _rows_spec(),
              pl.BlockSpec((1, _BM), lambda i: (0, i)),
              pl.BlockSpec((1, HID), lambda i: (0, 0)),
              pl.BlockSpec((HID, HID), lambda i: (0, 0))],
    out_specs=_rows_spec(),
)

_tc_head = pl.pallas_call(
    _tc_head_body,
    out_shape=jax.ShapeDtypeStruct((G, PROJ), jnp.float32),
    in_specs=[pl.BlockSpec((NC, NS, GP, HALF), lambda: (0, 0, 0, 0)),
              pl.BlockSpec((NC, NS, GP, HALF), lambda: (0, 0, 0, 0)),
              pl.BlockSpec((NC, NS, GP, 16), lambda: (0, 0, 0, 0)),
              pl.BlockSpec((2 * HID, HID), lambda: (0, 0)),
              pl.BlockSpec((1, HID), lambda: (0, 0)),
              pl.BlockSpec((HID, PROJ), lambda: (0, 0)),
              pl.BlockSpec((1, PROJ), lambda: (0, 0))],
    out_specs=pl.BlockSpec((G, PROJ), lambda: (0, 0)),
)


@jax.jit
def kernel(x, edge_index, node_class, batch, node_emb, w_in1, b_in1, w_in2,
           b_in2, gw1, gb1, gw2, gb2, gw3, gb3, wp1, bp1, wp2, bp2):
    src = jnp.concatenate([edge_index[0], jnp.zeros((EPAD - E,), jnp.int32)])
    dst = jnp.concatenate([edge_index[1], jnp.full((EPAD - E,), N, jnp.int32)])
    ncls = jnp.concatenate([node_class, jnp.zeros((NPAD - N,), jnp.int32)])
    batchp = jnp.concatenate([batch, jnp.full((NPAD - N,), G, jnp.int32)])
    xp = jnp.concatenate([x, jnp.zeros((NPAD - N, D_IN), jnp.float32)])
    gb3h = gb3.reshape(NC, HALF)

    t_tab = _tc_emb(node_emb, w_in1[D_IN:])
    erows, degp = _sc_gather_deg_call(t_tab, ncls, dst)
    g1, dinv = _tc_in(xp, erows, degp, w_in1[:D_IN], b_in1.reshape(1, HID),
                      w_in2, b_in2.reshape(1, HID), gw1)
    s1 = _sc_scatter_only(g1, src, dst)
    g2 = _tc_layer(s1, g1, dinv, gb1.reshape(1, HID), gw2)
    s2 = _sc_scatter_only(g2, src, dst)
    g3 = _tc_layer(s2, g2, dinv, gb2.reshape(1, HID), gw3)
    s3 = _sc_scatter_only(g3, src, dst)
    sump, maxp, cntp = _sc_pool_call(s3, g3, dinv, batchp, gb3h)
    z = _tc_head(sump, maxp, cntp, wp1, bp1.reshape(1, HID),
                 wp2, bp2.reshape(1, PROJ))
    return z
